# Initial kernel scaffold; baseline (speedup 1.0000x reference)
import jax
import jax.numpy as jnp
from jax.experimental import pallas as pl


def kernel(x, edge_index, emb_table, conv_W, conv_b, ln_w, ln_b, lin_W, lin_b, out_W, out_b):
    raise NotImplementedError("write your pallas kernel here")


# speedup vs baseline: 7.0808x; 7.0808x over previous
import functools

import jax
import jax.numpy as jnp
from jax import lax
from jax.experimental import pallas as pl
from jax.experimental.pallas import tpu as pltpu
from jax.experimental.pallas import tpu_sc as plsc

N = 10000
D = 128
NO_EMB = 20
L_LAYERS = 4

NC = 2
NS = 16
NTILES = NC * NS
CH = 128
NP = 10240
RPS = NP // NS
TRASH = N
RB = 1024
GRID = NP // RB

def _sc_scatter_body(hw, colr, rowr, zrows, out, colbuf, rowbuf, rows_v,
                     zbuf, acc, sem):
  c = lax.axis_index("c")
  s = lax.axis_index("s")
  wid = c * NS + s
  ep = colr.shape[0]
  per_tile = ep // NTILES
  pltpu.sync_copy(zrows, zbuf)
  for z in range(RPS // CH):
    pltpu.sync_copy(zbuf, acc.at[pl.ds(s * RPS + z * CH, CH)])
  plsc.subcore_barrier()

  base = wid * per_tile

  def body(k, carry):
    off = base + k * CH
    pltpu.sync_copy(colr.at[pl.ds(off, CH)], colbuf)
    pltpu.sync_copy(rowr.at[pl.ds(off, CH)], rowbuf)
    pltpu.async_copy(hw.at[colbuf], rows_v, sem).wait()
    pltpu.sync_copy(rows_v, acc.at[rowbuf], add=True)
    return carry

  lax.fori_loop(0, per_tile // CH, body, 0)
  plsc.subcore_barrier()
  pltpu.sync_copy(acc.at[pl.ds(s * RPS, RPS)],
                  out.at[c, pl.ds(s * RPS, RPS)])


@functools.cache
def _get_sc_scatter():
  mesh = plsc.VectorSubcoreMesh(core_axis_name="c", subcore_axis_name="s",
                                num_cores=NC, num_subcores=NS)
  return pl.kernel(
      _sc_scatter_body,
      out_type=jax.ShapeDtypeStruct((NC, NP, D), jnp.float32),
      mesh=mesh,
      scratch_types=[
          pltpu.VMEM((CH,), jnp.int32),
          pltpu.VMEM((CH,), jnp.int32),
          pltpu.VMEM((CH, D), jnp.float32),
          pltpu.VMEM((CH, D), jnp.float32),
          pltpu.VMEM_SHARED((NP, D), jnp.float32),
          pltpu.SemaphoreType.DMA,
      ],
  )


def _sc_deg_body(rowr, ones_h, zrows, out, rowbuf, ones_v, zbuf, acc):
  c = lax.axis_index("c")
  s = lax.axis_index("s")
  wid = c * NS + s
  ep = rowr.shape[0]
  per_tile = ep // NTILES
  pltpu.sync_copy(zrows, zbuf)
  for z in range(RPS // CH):
    pltpu.sync_copy(zbuf, acc.at[pl.ds(s * RPS + z * CH, CH)])
  pltpu.sync_copy(ones_h, ones_v)
  plsc.subcore_barrier()

  base = wid * per_tile

  def body(k, carry):
    off = base + k * CH
    pltpu.sync_copy(rowr.at[pl.ds(off, CH)], rowbuf)
    pltpu.sync_copy(ones_v, acc.at[rowbuf], add=True)
    return carry

  lax.fori_loop(0, per_tile // CH, body, 0)
  plsc.subcore_barrier()
  pltpu.sync_copy(acc.at[pl.ds(s * RPS, RPS)],
                  out.at[c, pl.ds(s * RPS, RPS)])


@functools.cache
def _get_sc_deg():
  mesh = plsc.VectorSubcoreMesh(core_axis_name="c", subcore_axis_name="s",
                                num_cores=NC, num_subcores=NS)
  return pl.kernel(
      _sc_deg_body,
      out_type=jax.ShapeDtypeStruct((NC, NP, D), jnp.float32),
      mesh=mesh,
      scratch_types=[
          pltpu.VMEM((CH,), jnp.int32),
          pltpu.VMEM((CH, D), jnp.float32),
          pltpu.VMEM((CH, D), jnp.float32),
          pltpu.VMEM_SHARED((NP, D), jnp.float32),
      ],
  )


def _tc_embed_body(x_ref, emb_ref, deg_ref, w0_ref, h_ref, hw0_ref,
                   dinv_ref):
  oh = (x_ref[:] == lax.broadcasted_iota(jnp.int32, (1, NO_EMB), 1))
  h = jnp.dot(oh.astype(jnp.float32), emb_ref[:],
              preferred_element_type=jnp.float32,
              precision=lax.Precision.HIGHEST)
  deg = deg_ref[0, :, 0:1] + deg_ref[1, :, 0:1] + 1.0
  dinv = lax.rsqrt(deg)
  h_ref[:] = h
  dinv_ref[:] = dinv
  hw0_ref[:] = jnp.dot(h, w0_ref[:], preferred_element_type=jnp.float32,
              precision=lax.Precision.HIGHEST) * dinv


_tc_embed = pl.pallas_call(
    _tc_embed_body,
    grid=(GRID,),
    in_specs=[
        pl.BlockSpec((RB, 1), lambda i: (i, 0)),
        pl.BlockSpec((NO_EMB, D), lambda i: (0, 0)),
        pl.BlockSpec((NC, RB, D), lambda i: (0, i, 0)),
        pl.BlockSpec((D, D), lambda i: (0, 0)),
    ],
    out_specs=[
        pl.BlockSpec((RB, D), lambda i: (i, 0)),
        pl.BlockSpec((RB, D), lambda i: (i, 0)),
        pl.BlockSpec((RB, 1), lambda i: (i, 0)),
    ],
    out_shape=[
        jax.ShapeDtypeStruct((NP, D), jnp.float32),
        jax.ShapeDtypeStruct((NP, D), jnp.float32),
        jax.ShapeDtypeStruct((NP, 1), jnp.float32),
    ],
)


def _tc_layer_body(h_ref, hwi_ref, s_ref, dinv_ref, b_ref, lnw_ref, lnb_ref,
                   wn_ref, hn_ref, hwn_ref):
  dinv = dinv_ref[:]
  conv = dinv * (s_ref[0] + s_ref[1] + hwi_ref[:]) + b_ref[:]
  hn = jnp.maximum(h_ref[:] + conv, 0.0)
  mu = jnp.mean(hn, axis=-1, keepdims=True)
  var = jnp.mean((hn - mu) ** 2, axis=-1, keepdims=True)
  hn = (hn - mu) * lax.rsqrt(var + 1e-5) * lnw_ref[:] + lnb_ref[:]
  hn_ref[:] = hn
  hwn_ref[:] = jnp.dot(hn, wn_ref[:], preferred_element_type=jnp.float32,
              precision=lax.Precision.HIGHEST) * dinv


_tc_layer = pl.pallas_call(
    _tc_layer_body,
    grid=(GRID,),
    in_specs=[
        pl.BlockSpec((RB, D), lambda i: (i, 0)),
        pl.BlockSpec((RB, D), lambda i: (i, 0)),
        pl.BlockSpec((NC, RB, D), lambda i: (0, i, 0)),
        pl.BlockSpec((RB, 1), lambda i: (i, 0)),
        pl.BlockSpec((1, D), lambda i: (0, 0)),
        pl.BlockSpec((1, D), lambda i: (0, 0)),
        pl.BlockSpec((1, D), lambda i: (0, 0)),
        pl.BlockSpec((D, D), lambda i: (0, 0)),
    ],
    out_specs=[
        pl.BlockSpec((RB, D), lambda i: (i, 0)),
        pl.BlockSpec((RB, D), lambda i: (i, 0)),
    ],
    out_shape=[
        jax.ShapeDtypeStruct((NP, D), jnp.float32),
        jax.ShapeDtypeStruct((NP, D), jnp.float32),
    ],
)


def _tc_final_body(h_ref, hwi_ref, s_ref, dinv_ref, b_ref, l0_ref, lb0_ref,
                   l1_ref, lb1_ref, ow_ref, ob_ref, out_ref):
  dinv = dinv_ref[:]
  conv = dinv * (s_ref[0] + s_ref[1] + hwi_ref[:]) + b_ref[:]
  hn = jnp.maximum(h_ref[:] + conv, 0.0)
  z = jnp.maximum(
      jnp.dot(hn, l0_ref[:], preferred_element_type=jnp.float32,
              precision=lax.Precision.HIGHEST) + lb0_ref[:],
      0.0)
  z = jnp.maximum(
      jnp.dot(z, l1_ref[:], preferred_element_type=jnp.float32,
              precision=lax.Precision.HIGHEST) + lb1_ref[:],
      0.0)
  out_ref[:] = (jnp.dot(z, ow_ref[:], preferred_element_type=jnp.float32,
              precision=lax.Precision.HIGHEST)
                + ob_ref[0, 0])


_tc_final = pl.pallas_call(
    _tc_final_body,
    grid=(GRID,),
    in_specs=[
        pl.BlockSpec((RB, D), lambda i: (i, 0)),
        pl.BlockSpec((RB, D), lambda i: (i, 0)),
        pl.BlockSpec((NC, RB, D), lambda i: (0, i, 0)),
        pl.BlockSpec((RB, 1), lambda i: (i, 0)),
        pl.BlockSpec((1, D), lambda i: (0, 0)),
        pl.BlockSpec((D, D), lambda i: (0, 0)),
        pl.BlockSpec((1, D), lambda i: (0, 0)),
        pl.BlockSpec((D, D), lambda i: (0, 0)),
        pl.BlockSpec((1, D), lambda i: (0, 0)),
        pl.BlockSpec((D, 1), lambda i: (0, 0)),
        pl.BlockSpec((1, 1), lambda i: (0, 0)),
    ],
    out_specs=[pl.BlockSpec((RB, 1), lambda i: (i, 0))],
    out_shape=[jax.ShapeDtypeStruct((NP, 1), jnp.float32)],
)


def kernel(x, edge_index, emb_table, conv_W, conv_b, ln_w, ln_b, lin_W,
           lin_b, out_W, out_b):
  e = edge_index.shape[1]
  ep = -(-e // (NTILES * CH)) * (NTILES * CH)
  pad = ep - e
  row = jnp.concatenate(
      [edge_index[0], jnp.full((pad,), TRASH, edge_index.dtype)])
  col = jnp.concatenate([edge_index[1], jnp.zeros((pad,), edge_index.dtype)])
  xp = jnp.concatenate([x, jnp.zeros((NP - N,), x.dtype)]).reshape(NP, 1)
  zrows = jnp.zeros((CH, D), jnp.float32)
  ones_ch = jnp.ones((CH, D), jnp.float32)

  degpart = _get_sc_deg()(row, ones_ch, zrows)
  h, hw, dinv = _tc_embed(xp, emb_table, degpart, conv_W[0])

  sc_scatter = _get_sc_scatter()
  for i in range(L_LAYERS):
    s = sc_scatter(hw, col, row, zrows)
    if i < L_LAYERS - 1:
      h, hw = _tc_layer(h, hw, s, dinv, conv_b[i].reshape(1, D),
                        ln_w[i].reshape(1, D), ln_b[i].reshape(1, D),
                        conv_W[i + 1])
    else:
      z, = _tc_final(h, hw, s, dinv, conv_b[i].reshape(1, D), lin_W[0],
                     lin_b[0].reshape(1, D), lin_W[1],
                     lin_b[1].reshape(1, D), out_W, out_b.reshape(1, 1))
  return z[:N, 0]
